# Initial kernel scaffold; baseline (speedup 1.0000x reference)
import jax
import jax.numpy as jnp
from jax.experimental import pallas as pl


def kernel(outputs, confidence, p_idx, u_idx, pse_n_idx, epoch):
    raise NotImplementedError("write your pallas kernel here")


# speedup vs baseline: 1.7840x; 1.7840x over previous
import functools

import jax
import jax.numpy as jnp
import numpy as np
from jax import lax
from jax.experimental import pallas as pl
from jax.experimental.pallas import tpu as pltpu
from jax.experimental.pallas import tpu_sc as plsc

_WARM_UP = 10
_N_ROWS = 16384
_N_IDX = 8192
_N_PSE = 2048
_NS = 16
_L = 16
_ROWS_PER_SC = _N_ROWS // _NS
_CHUNKS = _ROWS_PER_SC // _L
_IDX_PER_SC = _N_IDX // _NS
_IDX_BLK = 128
_PSE_PER_SC = _N_PSE // _NS


def _body(out_hbm, conf_hbm, pse_hbm, p_hbm, u_hbm, flag_hbm, res_hbm,
          out_v, conf_v, mask_v, l_v, zeros_v, pse_v, wval_v, pidx_v,
          uidx_v, vals_v, iota_v, flag_v, acc_v, mask_sh, l_sh, acc_sh,
          tmp_sh):
    sid = lax.axis_index("s")
    row0 = sid * _ROWS_PER_SC

    pltpu.sync_copy(out_hbm.at[pl.ds(row0, _ROWS_PER_SC)],
                    out_v.at[pl.ds(0, _ROWS_PER_SC)])
    pltpu.sync_copy(out_hbm.at[pl.ds(_N_ROWS + row0, _ROWS_PER_SC)],
                    out_v.at[pl.ds(_ROWS_PER_SC, _ROWS_PER_SC)])
    pltpu.sync_copy(conf_hbm.at[pl.ds(row0, _ROWS_PER_SC)],
                    conf_v.at[pl.ds(0, _ROWS_PER_SC)])
    pltpu.sync_copy(conf_hbm.at[pl.ds(_N_ROWS + row0, _ROWS_PER_SC)],
                    conf_v.at[pl.ds(_ROWS_PER_SC, _ROWS_PER_SC)])
    pltpu.sync_copy(pse_hbm.at[pl.ds(sid, 1)], pse_v)
    nblk = _IDX_PER_SC // _IDX_BLK
    pltpu.sync_copy(p_hbm.at[pl.ds(sid * nblk, nblk)], pidx_v)
    pltpu.sync_copy(u_hbm.at[pl.ds(sid * nblk, nblk)], uidx_v)
    pltpu.sync_copy(flag_hbm, flag_v)

    iota = lax.iota(jnp.int32, _L)
    iota_v[...] = iota
    zero16 = jnp.zeros((_L,), jnp.float32)

    def _zero(k, _):
        zeros_v[pl.ds(k * _L, _L)] = zero16
        return 0
    lax.fori_loop(0, _CHUNKS, _zero, 0)

    wvec = flag_v[...]
    for i in range(_PSE_PER_SC // _L):
        wval_v[pl.ds(i * _L, _L)] = wvec

    pltpu.sync_copy(zeros_v, mask_sh.at[pl.ds(row0, _ROWS_PER_SC)])

    @pl.when(sid == 0)
    def _():
        pltpu.sync_copy(zeros_v.at[pl.ds(0, _L)], acc_sh)

    plsc.subcore_barrier()

    pltpu.sync_copy(wval_v, mask_sh.at[pse_v.at[0]])
    plsc.subcore_barrier()

    pltpu.sync_copy(mask_sh.at[pl.ds(row0, _ROWS_PER_SC)], mask_v)

    c14 = jnp.float32(0.25)
    c13 = jnp.float32(1.0 / 3.0)
    c12 = jnp.float32(0.5)
    c1 = jnp.float32(1.0)

    def _dense(k, _):
        r0 = k * _L
        o0 = out_v[pl.ds(r0, _L)]
        o1 = out_v[pl.ds(_ROWS_PER_SC + r0, _L)]
        c0 = conf_v[pl.ds(r0, _L)]
        cc1 = conf_v[pl.ds(_ROWS_PER_SC + r0, _L)]
        e = jnp.exp(o1 - o0)
        inv = c1 / (c1 + e)
        s0 = inv
        s1 = e * inv
        g0 = s0 * (c1 + s0 * (c12 + s0 * (c13 + s0 * c14)))
        g1 = s1 * (c1 + s1 * (c12 + s1 * (c13 + s1 * c14)))
        l_orig = g1 * c0 + g0 * cc1
        m = mask_v[pl.ds(r0, _L)]
        l_v[pl.ds(r0, _L)] = l_orig + m * (g0 - l_orig)
        return 0
    lax.fori_loop(0, _CHUNKS, _dense, 0)

    pltpu.sync_copy(l_v, l_sh.at[pl.ds(row0, _ROWS_PER_SC)])
    plsc.subcore_barrier()

    for j in range(nblk):
        pltpu.sync_copy(l_sh.at[pidx_v.at[j]],
                        vals_v.at[pl.ds(j * _IDX_BLK, _IDX_BLK)])
        pltpu.sync_copy(l_sh.at[uidx_v.at[j]],
                        vals_v.at[pl.ds(_IDX_PER_SC + j * _IDX_BLK, _IDX_BLK)])

    def _sum(k, acc):
        return acc + vals_v[pl.ds(k * _L, _L)]
    acc = lax.fori_loop(0, (2 * _IDX_PER_SC) // _L, _sum, zero16)
    acc_v[...] = acc

    pltpu.sync_copy(acc_v, acc_sh.at[iota_v], add=True)
    plsc.subcore_barrier()

    @pl.when(sid == 0)
    def _():
        pltpu.sync_copy(acc_sh, flag_v)
        v = flag_v[...]
        for shift in (8, 4, 2, 1):
            acc_v[...] = v
            pltpu.sync_copy(acc_v, tmp_sh)
            iota_v[...] = (iota + shift) & (_L - 1)
            pltpu.sync_copy(tmp_sh.at[iota_v], flag_v)
            v = v + flag_v[...]
        acc_v[...] = v * jnp.float32(1.0 / (_N_IDX + 1e-8))
        pltpu.sync_copy(acc_v, res_hbm)


@jax.jit
def _cls_loss_sc(outputs, confidence, pse2d, p2d, u2d, flag):
    mesh = plsc.VectorSubcoreMesh(core_axis_name="c", subcore_axis_name="s")
    f32 = jnp.float32
    run = pl.kernel(
        _body,
        out_type=jax.ShapeDtypeStruct((_L,), f32),
        mesh=mesh,
        scratch_types=[
            pltpu.VMEM((2 * _ROWS_PER_SC,), f32),
            pltpu.VMEM((2 * _ROWS_PER_SC,), f32),
            pltpu.VMEM((_ROWS_PER_SC,), f32),
            pltpu.VMEM((_ROWS_PER_SC,), f32),
            pltpu.VMEM((_ROWS_PER_SC,), f32),
            pltpu.VMEM((1, _PSE_PER_SC), jnp.int32),
            pltpu.VMEM((_PSE_PER_SC,), f32),
            pltpu.VMEM((_IDX_PER_SC // _IDX_BLK, _IDX_BLK), jnp.int32),
            pltpu.VMEM((_IDX_PER_SC // _IDX_BLK, _IDX_BLK), jnp.int32),
            pltpu.VMEM((2 * _IDX_PER_SC,), f32),
            pltpu.VMEM((_L,), jnp.int32),
            pltpu.VMEM((_L,), f32),
            pltpu.VMEM((_L,), f32),
            pltpu.VMEM_SHARED((_N_ROWS,), f32),
            pltpu.VMEM_SHARED((_N_ROWS,), f32),
            pltpu.VMEM_SHARED((_L,), f32),
            pltpu.VMEM_SHARED((_L,), f32),
        ],
    )
    return run(outputs, confidence, pse2d, p2d, u2d, flag)


def kernel(outputs, confidence, p_idx, u_idx, pse_n_idx, epoch):
    w = jnp.where(jnp.asarray(epoch) >= _WARM_UP, 1.0, 0.0)
    flag = jnp.broadcast_to(w.astype(jnp.float32), (_L,))
    pse2d = pse_n_idx.reshape(_NS, _PSE_PER_SC)
    p2d = p_idx.reshape(_N_IDX // _IDX_BLK, _IDX_BLK)
    u2d = u_idx.reshape(_N_IDX // _IDX_BLK, _IDX_BLK)
    res = _cls_loss_sc(outputs.T.reshape(-1), confidence.T.reshape(-1),
                       pse2d, p2d, u2d, flag)
    return res[0]


# speedup vs baseline: 1.9089x; 1.0700x over previous
import functools

import jax
import jax.numpy as jnp
import numpy as np
from jax import lax
from jax.experimental import pallas as pl
from jax.experimental.pallas import tpu as pltpu
from jax.experimental.pallas import tpu_sc as plsc

_WARM_UP = 10
_N_ROWS = 16384
_N_IDX = 8192
_N_PSE = 2048
_NS = 16
_L = 16
_ROWS_PER_SC = _N_ROWS // _NS
_CHUNKS = _ROWS_PER_SC // _L
_IDX_PER_SC = _N_IDX // _NS
_IDX_BLK = 128
_PSE_PER_SC = _N_PSE // _NS


def _body(out_hbm, conf_hbm, pse_hbm, p_hbm, u_hbm, flag_hbm, res_hbm,
          out_v, conf_v, mask_v, l_v, zeros_v, pse_v, wval_v, pidx_v,
          uidx_v, vals_v, iota_v, flag_v, acc_v, mask_sh, l_sh, acc_sh,
          tmp_sh):
    sid = lax.axis_index("s")
    row0 = sid * _ROWS_PER_SC

    pltpu.sync_copy(out_hbm.at[pl.ds(row0, _ROWS_PER_SC)],
                    out_v.at[pl.ds(0, _ROWS_PER_SC)])
    pltpu.sync_copy(out_hbm.at[pl.ds(_N_ROWS + row0, _ROWS_PER_SC)],
                    out_v.at[pl.ds(_ROWS_PER_SC, _ROWS_PER_SC)])
    pltpu.sync_copy(conf_hbm.at[pl.ds(row0, _ROWS_PER_SC)],
                    conf_v.at[pl.ds(0, _ROWS_PER_SC)])
    pltpu.sync_copy(conf_hbm.at[pl.ds(_N_ROWS + row0, _ROWS_PER_SC)],
                    conf_v.at[pl.ds(_ROWS_PER_SC, _ROWS_PER_SC)])
    pltpu.sync_copy(pse_hbm.at[pl.ds(sid, 1)], pse_v)
    nblk = _IDX_PER_SC // _IDX_BLK
    pltpu.sync_copy(p_hbm.at[pl.ds(sid * nblk, nblk)], pidx_v)
    pltpu.sync_copy(u_hbm.at[pl.ds(sid * nblk, nblk)], uidx_v)
    pltpu.sync_copy(flag_hbm, flag_v)

    iota = lax.iota(jnp.int32, _L)
    iota_v[...] = iota
    zero16 = jnp.zeros((_L,), jnp.float32)

    def _zero(k, _):
        zeros_v[pl.ds(k * _L, _L)] = zero16
        return 0
    lax.fori_loop(0, _CHUNKS, _zero, 0)

    wvec = flag_v[...]
    for i in range(_PSE_PER_SC // _L):
        wval_v[pl.ds(i * _L, _L)] = wvec

    pltpu.sync_copy(zeros_v, mask_sh.at[pl.ds(row0, _ROWS_PER_SC)])

    @pl.when(sid == 0)
    def _():
        pltpu.sync_copy(zeros_v.at[pl.ds(0, _L)], acc_sh)

    plsc.subcore_barrier()

    pltpu.sync_copy(wval_v, mask_sh.at[pse_v.at[0]])
    plsc.subcore_barrier()

    pltpu.sync_copy(mask_sh.at[pl.ds(row0, _ROWS_PER_SC)], mask_v)

    c14 = jnp.float32(0.25)
    c13 = jnp.float32(1.0 / 3.0)
    c12 = jnp.float32(0.5)
    c1 = jnp.float32(1.0)

    def _dense(k, _):
        r0 = k * _L
        o0 = out_v[pl.ds(r0, _L)]
        o1 = out_v[pl.ds(_ROWS_PER_SC + r0, _L)]
        c0 = conf_v[pl.ds(r0, _L)]
        cc1 = conf_v[pl.ds(_ROWS_PER_SC + r0, _L)]
        e = jnp.exp(o1 - o0)
        inv = c1 / (c1 + e)
        s0 = inv
        s1 = e * inv
        g0 = s0 * (c1 + s0 * (c12 + s0 * (c13 + s0 * c14)))
        g1 = s1 * (c1 + s1 * (c12 + s1 * (c13 + s1 * c14)))
        l_orig = g1 * c0 + g0 * cc1
        m = mask_v[pl.ds(r0, _L)]
        l_v[pl.ds(r0, _L)] = l_orig + m * (g0 - l_orig)
        return 0
    lax.fori_loop(0, _CHUNKS, _dense, 0)

    pltpu.sync_copy(l_v, l_sh.at[pl.ds(row0, _ROWS_PER_SC)])
    plsc.subcore_barrier()

    for j in range(nblk):
        pltpu.sync_copy(l_sh.at[pidx_v.at[j]],
                        vals_v.at[pl.ds(j * _IDX_BLK, _IDX_BLK)])
        pltpu.sync_copy(l_sh.at[uidx_v.at[j]],
                        vals_v.at[pl.ds(_IDX_PER_SC + j * _IDX_BLK, _IDX_BLK)])

    def _sum(k, acc):
        return acc + vals_v[pl.ds(k * _L, _L)]
    acc = lax.fori_loop(0, (2 * _IDX_PER_SC) // _L, _sum, zero16)
    acc_v[...] = acc

    pltpu.sync_copy(acc_v, acc_sh.at[iota_v], add=True)
    plsc.subcore_barrier()

    @pl.when(sid == 0)
    def _():
        pltpu.sync_copy(acc_sh, flag_v)
        v = flag_v[...]
        for shift in (8, 4, 2, 1):
            acc_v[...] = v
            pltpu.sync_copy(acc_v, tmp_sh)
            iota_v[...] = (iota + shift) & (_L - 1)
            pltpu.sync_copy(tmp_sh.at[iota_v], flag_v)
            v = v + flag_v[...]
        acc_v[...] = v * jnp.float32(1.0 / (_N_IDX + 1e-8))
        pltpu.sync_copy(acc_v, res_hbm)


@jax.jit
def _cls_loss_sc(outputs, confidence, pse2d, p2d, u2d, flag):
    mesh = plsc.VectorSubcoreMesh(core_axis_name="c", subcore_axis_name="s",
                                  num_cores=1)
    f32 = jnp.float32
    run = pl.kernel(
        _body,
        out_type=jax.ShapeDtypeStruct((_L,), f32),
        mesh=mesh,
        scratch_types=[
            pltpu.VMEM((2 * _ROWS_PER_SC,), f32),
            pltpu.VMEM((2 * _ROWS_PER_SC,), f32),
            pltpu.VMEM((_ROWS_PER_SC,), f32),
            pltpu.VMEM((_ROWS_PER_SC,), f32),
            pltpu.VMEM((_ROWS_PER_SC,), f32),
            pltpu.VMEM((1, _PSE_PER_SC), jnp.int32),
            pltpu.VMEM((_PSE_PER_SC,), f32),
            pltpu.VMEM((_IDX_PER_SC // _IDX_BLK, _IDX_BLK), jnp.int32),
            pltpu.VMEM((_IDX_PER_SC // _IDX_BLK, _IDX_BLK), jnp.int32),
            pltpu.VMEM((2 * _IDX_PER_SC,), f32),
            pltpu.VMEM((_L,), jnp.int32),
            pltpu.VMEM((_L,), f32),
            pltpu.VMEM((_L,), f32),
            pltpu.VMEM_SHARED((_N_ROWS,), f32),
            pltpu.VMEM_SHARED((_N_ROWS,), f32),
            pltpu.VMEM_SHARED((_L,), f32),
            pltpu.VMEM_SHARED((_L,), f32),
        ],
    )
    return run(outputs, confidence, pse2d, p2d, u2d, flag)


def kernel(outputs, confidence, p_idx, u_idx, pse_n_idx, epoch):
    w = jnp.where(jnp.asarray(epoch) >= _WARM_UP, 1.0, 0.0)
    flag = jnp.broadcast_to(w.astype(jnp.float32), (_L,))
    pse2d = pse_n_idx.reshape(_NS, _PSE_PER_SC)
    p2d = p_idx.reshape(_N_IDX // _IDX_BLK, _IDX_BLK)
    u2d = u_idx.reshape(_N_IDX // _IDX_BLK, _IDX_BLK)
    res = _cls_loss_sc(outputs.T.reshape(-1), confidence.T.reshape(-1),
                       pse2d, p2d, u2d, flag)
    return res[0]


# speedup vs baseline: 2.2917x; 1.2006x over previous
import functools

import jax
import jax.numpy as jnp
import numpy as np
from jax import lax
from jax.experimental import pallas as pl
from jax.experimental.pallas import tpu as pltpu
from jax.experimental.pallas import tpu_sc as plsc

_WARM_UP = 10
_N_ROWS = 16384
_N_IDX = 8192
_N_PSE = 2048
_NS = 16
_L = 16
_ROWS_PER_SC = _N_ROWS // _NS
_CHUNKS = _ROWS_PER_SC // _L
_IDX_PER_SC = _N_IDX // _NS
_IDX_BLK = 128
_PSE_PER_SC = _N_PSE // _NS
_NBLK = _IDX_PER_SC // _IDX_BLK


def _body(out_hbm, conf_hbm, pse_hbm, p_hbm, u_hbm, flag_hbm, res_hbm,
          out_v, conf_v, l_v, lalt_v, pse_v, pidx_v, uidx_v, vals_v,
          plo_v, pla_v, scat_v, flag_v, acc_v, iota_v,
          l_sh, lalt_sh, acc_sh, tmp_sh, sem):
    sid = lax.axis_index("s")
    row0 = sid * _ROWS_PER_SC

    ds = [
        pltpu.async_copy(out_hbm.at[pl.ds(2 * row0, 2 * _ROWS_PER_SC)],
                         out_v, sem),
        pltpu.async_copy(conf_hbm.at[pl.ds(2 * row0, 2 * _ROWS_PER_SC)],
                         conf_v, sem),
        pltpu.async_copy(pse_hbm.at[pl.ds(sid, 1)], pse_v, sem),
        pltpu.async_copy(p_hbm.at[pl.ds(sid * _NBLK, _NBLK)], pidx_v, sem),
        pltpu.async_copy(u_hbm.at[pl.ds(sid * _NBLK, _NBLK)], uidx_v, sem),
        pltpu.async_copy(flag_hbm, flag_v, sem),
    ]

    iota = lax.iota(jnp.int32, _L)
    iota_v[...] = iota
    zero16 = jnp.zeros((_L,), jnp.float32)
    acc_v[...] = zero16

    @pl.when(sid == 0)
    def _():
        pltpu.sync_copy(acc_v, acc_sh)

    for d in ds:
        d.wait()

    c14 = jnp.float32(0.25)
    c13 = jnp.float32(1.0 / 3.0)
    c12 = jnp.float32(0.5)
    c1 = jnp.float32(1.0)

    def _dense(k, _):
        r0 = k * _L
        o0 = out_v[pl.ds(r0, _L)]
        o1 = out_v[pl.ds(_ROWS_PER_SC + r0, _L)]
        c0 = conf_v[pl.ds(r0, _L)]
        cc1 = conf_v[pl.ds(_ROWS_PER_SC + r0, _L)]
        e = jnp.exp(o1 - o0)
        inv = c1 / (c1 + e)
        s0 = inv
        s1 = e * inv
        g0 = s0 * (c1 + s0 * (c12 + s0 * (c13 + s0 * c14)))
        g1 = s1 * (c1 + s1 * (c12 + s1 * (c13 + s1 * c14)))
        l_v[pl.ds(r0, _L)] = g1 * c0 + g0 * cc1
        lalt_v[pl.ds(r0, _L)] = g0
        return 0
    lax.fori_loop(0, _CHUNKS, _dense, 0)

    dp = [
        pltpu.async_copy(l_v, l_sh.at[pl.ds(row0, _ROWS_PER_SC)], sem),
        pltpu.async_copy(lalt_v, lalt_sh.at[pl.ds(row0, _ROWS_PER_SC)], sem),
    ]
    for d in dp:
        d.wait()
    plsc.subcore_barrier()

    dg = [
        pltpu.async_copy(l_sh.at[pse_v.at[0]], plo_v, sem),
        pltpu.async_copy(lalt_sh.at[pse_v.at[0]], pla_v, sem),
    ]
    for d in dg:
        d.wait()
    w = flag_v[...]
    for i in range(_PSE_PER_SC // _L):
        lo = plo_v[pl.ds(i * _L, _L)]
        la = pla_v[pl.ds(i * _L, _L)]
        scat_v[pl.ds(i * _L, _L)] = lo + w * (la - lo)
    pltpu.sync_copy(scat_v, l_sh.at[pse_v.at[0]])
    plsc.subcore_barrier()

    dv = []
    for j in range(_NBLK):
        dv.append(pltpu.async_copy(
            l_sh.at[pidx_v.at[j]],
            vals_v.at[pl.ds(j * _IDX_BLK, _IDX_BLK)], sem))
        dv.append(pltpu.async_copy(
            l_sh.at[uidx_v.at[j]],
            vals_v.at[pl.ds(_IDX_PER_SC + j * _IDX_BLK, _IDX_BLK)], sem))
    for d in dv:
        d.wait()

    def _sum(k, acc):
        return acc + vals_v[pl.ds(k * _L, _L)]
    acc = lax.fori_loop(0, (2 * _IDX_PER_SC) // _L, _sum, zero16)
    acc_v[...] = acc

    pltpu.sync_copy(acc_v, acc_sh.at[iota_v], add=True)
    plsc.subcore_barrier()

    @pl.when(sid == 0)
    def _():
        pltpu.sync_copy(acc_sh, flag_v)
        v = flag_v[...]
        for shift in (8, 4, 2, 1):
            acc_v[...] = v
            pltpu.sync_copy(acc_v, tmp_sh)
            iota_v[...] = (iota + shift) & (_L - 1)
            pltpu.sync_copy(tmp_sh.at[iota_v], flag_v)
            v = v + flag_v[...]
        acc_v[...] = v * jnp.float32(1.0 / (_N_IDX + 1e-8))
        pltpu.sync_copy(acc_v, res_hbm)


@jax.jit
def _cls_loss_sc(outputs, confidence, pse2d, p2d, u2d, flag):
    mesh = plsc.VectorSubcoreMesh(core_axis_name="c", subcore_axis_name="s",
                                  num_cores=1)
    f32 = jnp.float32
    run = pl.kernel(
        _body,
        out_type=jax.ShapeDtypeStruct((_L,), f32),
        mesh=mesh,
        scratch_types=[
            pltpu.VMEM((2 * _ROWS_PER_SC,), f32),
            pltpu.VMEM((2 * _ROWS_PER_SC,), f32),
            pltpu.VMEM((_ROWS_PER_SC,), f32),
            pltpu.VMEM((_ROWS_PER_SC,), f32),
            pltpu.VMEM((1, _PSE_PER_SC), jnp.int32),
            pltpu.VMEM((_NBLK, _IDX_BLK), jnp.int32),
            pltpu.VMEM((_NBLK, _IDX_BLK), jnp.int32),
            pltpu.VMEM((2 * _IDX_PER_SC,), f32),
            pltpu.VMEM((_PSE_PER_SC,), f32),
            pltpu.VMEM((_PSE_PER_SC,), f32),
            pltpu.VMEM((_PSE_PER_SC,), f32),
            pltpu.VMEM((_L,), f32),
            pltpu.VMEM((_L,), f32),
            pltpu.VMEM((_L,), jnp.int32),
            pltpu.VMEM_SHARED((_N_ROWS,), f32),
            pltpu.VMEM_SHARED((_N_ROWS,), f32),
            pltpu.VMEM_SHARED((_L,), f32),
            pltpu.VMEM_SHARED((_L,), f32),
            pltpu.SemaphoreType.DMA,
        ],
    )
    return run(outputs, confidence, pse2d, p2d, u2d, flag)


def kernel(outputs, confidence, p_idx, u_idx, pse_n_idx, epoch):
    w = jnp.where(jnp.asarray(epoch) >= _WARM_UP, 1.0, 0.0)
    flag = jnp.broadcast_to(w.astype(jnp.float32), (_L,))
    o1d = outputs.T.reshape(2, _NS, _ROWS_PER_SC).swapaxes(0, 1).reshape(-1)
    c1d = confidence.T.reshape(2, _NS, _ROWS_PER_SC).swapaxes(0, 1).reshape(-1)
    pse2d = pse_n_idx.reshape(_NS, _PSE_PER_SC)
    p2d = p_idx.reshape(_N_IDX // _IDX_BLK, _IDX_BLK)
    u2d = u_idx.reshape(_N_IDX // _IDX_BLK, _IDX_BLK)
    res = _cls_loss_sc(o1d, c1d, pse2d, p2d, u2d, flag)
    return res[0]
